# baseline (speedup 1.0000x reference)
import math

import jax
import jax.numpy as jnp
from jax.experimental import pallas as pl
from jax.experimental.pallas import tpu as pltpu
from jax.experimental.pallas import tpu_sc as plsc

_CUTOFF = 10.0
_EDGE_BLOCK = 2560
_GATHER_WIN = 128
_N_CHUNKS = 5


def _cos_body(d_ref, o_ref):
    o_ref[...] = jnp.transpose(
        0.5 * (jnp.cos(d_ref[...] * (math.pi / _CUTOFF)) + 1.0), (0, 2, 1))


def _edge_body(c_ref, demb_ref, g_ref, lint_ref, w1t_ref, b1_ref, w2t_ref,
               b2_ref, w3t_ref, b3_ref, owt_ref, ob_ref, o_ref):
    bf = jnp.bfloat16
    h = jax.lax.dot_general(
        demb_ref[...].astype(bf), w1t_ref[...].astype(bf),
        (((0,), (0,)), ((), ())),
        preferred_element_type=jnp.float32) + b1_ref[...]
    h = jnp.where(h > 0, h, 0.015 * h)
    h = jnp.dot(h.astype(bf), w2t_ref[...].astype(bf),
                preferred_element_type=jnp.float32) + b2_ref[...]
    h = jnp.where(h > 0, h, 0.015 * h)
    w = jnp.dot(h.astype(bf), w3t_ref[...].astype(bf),
                preferred_element_type=jnp.float32) + b3_ref[...]
    ct = c_ref[0]
    scaled = [w[t * 128:(t + 1) * 128, :] * ct[:, t:t + 1]
              for t in range(_EDGE_BLOCK // 128)]
    vp = jnp.dot(g_ref[...].astype(bf), lint_ref[...].astype(bf),
                 preferred_element_type=jnp.float32)
    e = vp * jnp.concatenate(scaled, axis=0)
    o_ref[...] = jnp.dot(e.astype(bf), owt_ref[...].astype(bf),
                         preferred_element_type=jnp.float32) + ob_ref[...] + e


def _edge_body_chain(c_ref, demb_ref, g_ref, lint_ref, w1t_ref, b1_ref,
                     w2t_ref, b2_ref, w3t_ref, b3_ref, owt_ref, ob_ref,
                     buf_ref, o_ref):
    del buf_ref
    _edge_body(c_ref, demb_ref, g_ref, lint_ref, w1t_ref, b1_ref, w2t_ref,
               b2_ref, w3t_ref, b3_ref, owt_ref, ob_ref, o_ref)


def _sc_gather(table, idx):
    n_win, win = idx.shape
    feat = table.shape[1]
    mesh = plsc.VectorSubcoreMesh(core_axis_name="core",
                                  subcore_axis_name="subcore")

    @pl.kernel(
        out_type=jax.ShapeDtypeStruct((n_win * win, feat), table.dtype),
        mesh=mesh,
    )
    def gather_kernel(x_hbm, i_hbm, o_hbm):
        def body(i_vmem, o_vmem):
            pltpu.sync_copy(x_hbm.at[i_vmem.at[0]], o_vmem)

        pltpu.emit_pipeline(
            body,
            grid=(n_win,),
            in_specs=[pl.BlockSpec((1, win), index_map=lambda i: (i, 0))],
            out_specs=[pl.BlockSpec((win, feat), index_map=lambda i: (i, 0))],
            core_axis_name=("core", "subcore"),
            dimension_semantics=(pltpu.PARALLEL,),
        )(i_hbm, o_hbm)

    return gather_kernel(table, idx)


def kernel(v, dist, dist_emb, edge_index, lin_w, mlp_w1, mlp_b1, mlp_w2,
           mlp_b2, mlp_w3, mlp_b3, out_w, out_b):
    n_nodes, hidden = v.shape
    n_edges = dist.shape[0]
    filters = lin_w.shape[0]

    nb = n_edges // _EDGE_BLOCK
    sub = _EDGE_BLOCK // 128
    c_t = pl.pallas_call(
        _cos_body,
        out_shape=jax.ShapeDtypeStruct((nb, 128, sub), jnp.float32),
    )(dist.reshape(nb, sub, 128))

    j2d = edge_index[0].reshape(n_edges // _GATHER_WIN, _GATHER_WIN)
    blocks_per_chunk = nb // _N_CHUNKS
    wins_per_chunk = j2d.shape[0] // _N_CHUNKS
    weights = (
        lin_w.T, mlp_w1.T, mlp_b1.reshape(1, filters),
        mlp_w2.T, mlp_b2.reshape(1, filters),
        mlp_w3.T, mlp_b3.reshape(1, filters),
        out_w.T, out_b.reshape(1, filters),
    )
    full = lambda i: (0, 0)
    weight_specs = [
        pl.BlockSpec((filters, filters), full),
        pl.BlockSpec((dist_emb.shape[1], filters), full),
        pl.BlockSpec((1, filters), full),
        pl.BlockSpec((filters, filters), full),
        pl.BlockSpec((1, filters), full),
        pl.BlockSpec((filters, filters), full),
        pl.BlockSpec((1, filters), full),
        pl.BlockSpec((filters, filters), full),
        pl.BlockSpec((1, filters), full),
    ]

    buf = None
    for k in range(_N_CHUNKS):
        g_k = _sc_gather(
            v, j2d[k * wins_per_chunk:(k + 1) * wins_per_chunk])
        base = k * blocks_per_chunk

        def off3(i, base=base):
            return (base + i, 0, 0)

        def off2(i, base=base):
            return (base + i, 0)

        def off2t(i, base=base):
            return (0, base + i)

        in_specs = [
            pl.BlockSpec((1, 128, sub), off3),
            pl.BlockSpec((dist_emb.shape[1], _EDGE_BLOCK), off2t),
            pl.BlockSpec((_EDGE_BLOCK, filters), lambda i: (i, 0)),
        ] + list(weight_specs)
        args = (c_t, dist_emb.T, g_k) + weights
        if buf is None:
            buf = pl.pallas_call(
                _edge_body,
                grid=(blocks_per_chunk,),
                in_specs=in_specs,
                out_specs=pl.BlockSpec((_EDGE_BLOCK, filters), off2),
                out_shape=jax.ShapeDtypeStruct((n_edges, filters),
                                               jnp.float32),
            )(*args)
        else:
            buf = pl.pallas_call(
                _edge_body_chain,
                grid=(blocks_per_chunk,),
                in_specs=in_specs + [pl.BlockSpec(memory_space=pl.ANY)],
                out_specs=pl.BlockSpec((_EDGE_BLOCK, filters), off2),
                out_shape=jax.ShapeDtypeStruct((n_edges, filters),
                                               jnp.float32),
                input_output_aliases={12: 0},
            )(*args, buf)
    return buf


# speedup vs baseline: 1.0985x; 1.0985x over previous
import math

import jax
import jax.numpy as jnp
from jax.experimental import pallas as pl
from jax.experimental.pallas import tpu as pltpu
from jax.experimental.pallas import tpu_sc as plsc

_CUTOFF = 10.0
_EDGE_BLOCK = 6400
_GATHER_WIN = 128
_N_CHUNKS = 5


def _cos_body(d_ref, o_ref):
    o_ref[...] = jnp.transpose(
        0.5 * (jnp.cos(d_ref[...] * (math.pi / _CUTOFF)) + 1.0), (0, 2, 1))


def _edge_body(c_ref, demb_ref, g_ref, lint_ref, w1t_ref, b1_ref, w2t_ref,
               b2_ref, w3t_ref, b3_ref, owt_ref, ob_ref, o_ref):
    bf = jnp.bfloat16
    h = jax.lax.dot_general(
        demb_ref[...].astype(bf), w1t_ref[...].astype(bf),
        (((0,), (0,)), ((), ())),
        preferred_element_type=jnp.float32) + b1_ref[...]
    h = jnp.where(h > 0, h, 0.015 * h)
    h = jnp.dot(h.astype(bf), w2t_ref[...].astype(bf),
                preferred_element_type=jnp.float32) + b2_ref[...]
    h = jnp.where(h > 0, h, 0.015 * h)
    w = jnp.dot(h.astype(bf), w3t_ref[...].astype(bf),
                preferred_element_type=jnp.float32) + b3_ref[...]
    ct = c_ref[0]
    scaled = [w[t * 128:(t + 1) * 128, :] * ct[:, t:t + 1]
              for t in range(_EDGE_BLOCK // 128)]
    vp = jnp.dot(g_ref[...].astype(bf), lint_ref[...].astype(bf),
                 preferred_element_type=jnp.float32)
    e = vp * jnp.concatenate(scaled, axis=0)
    o_ref[...] = jnp.dot(e.astype(bf), owt_ref[...].astype(bf),
                         preferred_element_type=jnp.float32) + ob_ref[...] + e


def _edge_body_chain(c_ref, demb_ref, g_ref, lint_ref, w1t_ref, b1_ref,
                     w2t_ref, b2_ref, w3t_ref, b3_ref, owt_ref, ob_ref,
                     buf_ref, o_ref):
    del buf_ref
    _edge_body(c_ref, demb_ref, g_ref, lint_ref, w1t_ref, b1_ref, w2t_ref,
               b2_ref, w3t_ref, b3_ref, owt_ref, ob_ref, o_ref)


def _sc_gather(table, idx):
    n_win, win = idx.shape
    feat = table.shape[1]
    mesh = plsc.VectorSubcoreMesh(core_axis_name="core",
                                  subcore_axis_name="subcore")

    @pl.kernel(
        out_type=jax.ShapeDtypeStruct((n_win * win, feat), table.dtype),
        mesh=mesh,
    )
    def gather_kernel(x_hbm, i_hbm, o_hbm):
        def body(i_vmem, o_vmem):
            pltpu.sync_copy(x_hbm.at[i_vmem.at[0]], o_vmem)

        pltpu.emit_pipeline(
            body,
            grid=(n_win,),
            in_specs=[pl.BlockSpec((1, win), index_map=lambda i: (i, 0))],
            out_specs=[pl.BlockSpec((win, feat), index_map=lambda i: (i, 0))],
            core_axis_name=("core", "subcore"),
            dimension_semantics=(pltpu.PARALLEL,),
        )(i_hbm, o_hbm)

    return gather_kernel(table, idx)


def kernel(v, dist, dist_emb, edge_index, lin_w, mlp_w1, mlp_b1, mlp_w2,
           mlp_b2, mlp_w3, mlp_b3, out_w, out_b):
    n_nodes, hidden = v.shape
    n_edges = dist.shape[0]
    filters = lin_w.shape[0]

    nb = n_edges // _EDGE_BLOCK
    sub = _EDGE_BLOCK // 128
    c_t = pl.pallas_call(
        _cos_body,
        out_shape=jax.ShapeDtypeStruct((nb, 128, sub), jnp.float32),
    )(dist.reshape(nb, sub, 128))

    j2d = edge_index[0].reshape(n_edges // _GATHER_WIN, _GATHER_WIN)
    blocks_per_chunk = nb // _N_CHUNKS
    wins_per_chunk = j2d.shape[0] // _N_CHUNKS
    weights = (
        lin_w.T, mlp_w1.T, mlp_b1.reshape(1, filters),
        mlp_w2.T, mlp_b2.reshape(1, filters),
        mlp_w3.T, mlp_b3.reshape(1, filters),
        out_w.T, out_b.reshape(1, filters),
    )
    full = lambda i: (0, 0)
    weight_specs = [
        pl.BlockSpec((filters, filters), full),
        pl.BlockSpec((dist_emb.shape[1], filters), full),
        pl.BlockSpec((1, filters), full),
        pl.BlockSpec((filters, filters), full),
        pl.BlockSpec((1, filters), full),
        pl.BlockSpec((filters, filters), full),
        pl.BlockSpec((1, filters), full),
        pl.BlockSpec((filters, filters), full),
        pl.BlockSpec((1, filters), full),
    ]

    buf = None
    for k in range(_N_CHUNKS):
        g_k = _sc_gather(
            v, j2d[k * wins_per_chunk:(k + 1) * wins_per_chunk])
        base = k * blocks_per_chunk

        def off3(i, base=base):
            return (base + i, 0, 0)

        def off2(i, base=base):
            return (base + i, 0)

        def off2t(i, base=base):
            return (0, base + i)

        in_specs = [
            pl.BlockSpec((1, 128, sub), off3),
            pl.BlockSpec((dist_emb.shape[1], _EDGE_BLOCK), off2t),
            pl.BlockSpec((_EDGE_BLOCK, filters), lambda i: (i, 0)),
        ] + list(weight_specs)
        args = (c_t, dist_emb.T, g_k) + weights
        if buf is None:
            buf = pl.pallas_call(
                _edge_body,
                grid=(blocks_per_chunk,),
                in_specs=in_specs,
                out_specs=pl.BlockSpec((_EDGE_BLOCK, filters), off2),
                out_shape=jax.ShapeDtypeStruct((n_edges, filters),
                                               jnp.float32),
            )(*args)
        else:
            buf = pl.pallas_call(
                _edge_body_chain,
                grid=(blocks_per_chunk,),
                in_specs=in_specs + [pl.BlockSpec(memory_space=pl.ANY)],
                out_specs=pl.BlockSpec((_EDGE_BLOCK, filters), off2),
                out_shape=jax.ShapeDtypeStruct((n_edges, filters),
                                               jnp.float32),
                input_output_aliases={12: 0},
            )(*args, buf)
    return buf


# speedup vs baseline: 1.1236x; 1.0229x over previous
import math

import jax
import jax.numpy as jnp
from jax.experimental import pallas as pl
from jax.experimental.pallas import tpu as pltpu
from jax.experimental.pallas import tpu_sc as plsc

_CUTOFF = 10.0
_EDGE_BLOCK = 12800
_GATHER_WIN = 128
_N_CHUNKS = 5


def _cos_body(d_ref, o_ref):
    o_ref[...] = jnp.transpose(
        0.5 * (jnp.cos(d_ref[...] * (math.pi / _CUTOFF)) + 1.0), (0, 2, 1))


def _edge_body(c_ref, demb_ref, g_ref, lint_ref, w1t_ref, b1_ref, w2t_ref,
               b2_ref, w3t_ref, b3_ref, owt_ref, ob_ref, o_ref):
    bf = jnp.bfloat16
    h = jax.lax.dot_general(
        demb_ref[...].astype(bf), w1t_ref[...].astype(bf),
        (((0,), (0,)), ((), ())),
        preferred_element_type=jnp.float32) + b1_ref[...]
    h = jnp.where(h > 0, h, 0.015 * h)
    h = jnp.dot(h.astype(bf), w2t_ref[...].astype(bf),
                preferred_element_type=jnp.float32) + b2_ref[...]
    h = jnp.where(h > 0, h, 0.015 * h)
    w = jnp.dot(h.astype(bf), w3t_ref[...].astype(bf),
                preferred_element_type=jnp.float32) + b3_ref[...]
    ct = c_ref[0]
    scaled = [w[t * 128:(t + 1) * 128, :] * ct[:, t:t + 1]
              for t in range(_EDGE_BLOCK // 128)]
    vp = jnp.dot(g_ref[...].astype(bf), lint_ref[...].astype(bf),
                 preferred_element_type=jnp.float32)
    e = vp * jnp.concatenate(scaled, axis=0)
    o_ref[...] = jnp.dot(e.astype(bf), owt_ref[...].astype(bf),
                         preferred_element_type=jnp.float32) + ob_ref[...] + e


def _edge_body_chain(c_ref, demb_ref, g_ref, lint_ref, w1t_ref, b1_ref,
                     w2t_ref, b2_ref, w3t_ref, b3_ref, owt_ref, ob_ref,
                     buf_ref, o_ref):
    del buf_ref
    _edge_body(c_ref, demb_ref, g_ref, lint_ref, w1t_ref, b1_ref, w2t_ref,
               b2_ref, w3t_ref, b3_ref, owt_ref, ob_ref, o_ref)


def _sc_gather(table, idx):
    n_win, win = idx.shape
    feat = table.shape[1]
    mesh = plsc.VectorSubcoreMesh(core_axis_name="core",
                                  subcore_axis_name="subcore")

    @pl.kernel(
        out_type=jax.ShapeDtypeStruct((n_win * win, feat), table.dtype),
        mesh=mesh,
    )
    def gather_kernel(x_hbm, i_hbm, o_hbm):
        def body(i_vmem, o_vmem):
            pltpu.sync_copy(x_hbm.at[i_vmem.at[0]], o_vmem)

        pltpu.emit_pipeline(
            body,
            grid=(n_win,),
            in_specs=[pl.BlockSpec((1, win), index_map=lambda i: (i, 0))],
            out_specs=[pl.BlockSpec((win, feat), index_map=lambda i: (i, 0))],
            core_axis_name=("core", "subcore"),
            dimension_semantics=(pltpu.PARALLEL,),
        )(i_hbm, o_hbm)

    return gather_kernel(table, idx)


def kernel(v, dist, dist_emb, edge_index, lin_w, mlp_w1, mlp_b1, mlp_w2,
           mlp_b2, mlp_w3, mlp_b3, out_w, out_b):
    n_nodes, hidden = v.shape
    n_edges = dist.shape[0]
    filters = lin_w.shape[0]

    nb = n_edges // _EDGE_BLOCK
    sub = _EDGE_BLOCK // 128
    c_t = pl.pallas_call(
        _cos_body,
        out_shape=jax.ShapeDtypeStruct((nb, 128, sub), jnp.float32),
    )(dist.reshape(nb, sub, 128))

    j2d = edge_index[0].reshape(n_edges // _GATHER_WIN, _GATHER_WIN)
    blocks_per_chunk = nb // _N_CHUNKS
    wins_per_chunk = j2d.shape[0] // _N_CHUNKS
    weights = (
        lin_w.T, mlp_w1.T, mlp_b1.reshape(1, filters),
        mlp_w2.T, mlp_b2.reshape(1, filters),
        mlp_w3.T, mlp_b3.reshape(1, filters),
        out_w.T, out_b.reshape(1, filters),
    )
    full = lambda i: (0, 0)
    weight_specs = [
        pl.BlockSpec((filters, filters), full),
        pl.BlockSpec((dist_emb.shape[1], filters), full),
        pl.BlockSpec((1, filters), full),
        pl.BlockSpec((filters, filters), full),
        pl.BlockSpec((1, filters), full),
        pl.BlockSpec((filters, filters), full),
        pl.BlockSpec((1, filters), full),
        pl.BlockSpec((filters, filters), full),
        pl.BlockSpec((1, filters), full),
    ]

    buf = None
    for k in range(_N_CHUNKS):
        g_k = _sc_gather(
            v, j2d[k * wins_per_chunk:(k + 1) * wins_per_chunk])
        base = k * blocks_per_chunk

        def off3(i, base=base):
            return (base + i, 0, 0)

        def off2(i, base=base):
            return (base + i, 0)

        def off2t(i, base=base):
            return (0, base + i)

        in_specs = [
            pl.BlockSpec((1, 128, sub), off3),
            pl.BlockSpec((dist_emb.shape[1], _EDGE_BLOCK), off2t),
            pl.BlockSpec((_EDGE_BLOCK, filters), lambda i: (i, 0)),
        ] + list(weight_specs)
        args = (c_t, dist_emb.T, g_k) + weights
        if buf is None:
            buf = pl.pallas_call(
                _edge_body,
                grid=(blocks_per_chunk,),
                in_specs=in_specs,
                out_specs=pl.BlockSpec((_EDGE_BLOCK, filters), off2),
                out_shape=jax.ShapeDtypeStruct((n_edges, filters),
                                               jnp.float32),
            )(*args)
        else:
            buf = pl.pallas_call(
                _edge_body_chain,
                grid=(blocks_per_chunk,),
                in_specs=in_specs + [pl.BlockSpec(memory_space=pl.ANY)],
                out_specs=pl.BlockSpec((_EDGE_BLOCK, filters), off2),
                out_shape=jax.ShapeDtypeStruct((n_edges, filters),
                                               jnp.float32),
                input_output_aliases={12: 0},
            )(*args, buf)
    return buf
